# baseline (speedup 1.0000x reference)
import functools

import jax
import jax.numpy as jnp
from jax import lax
from jax.experimental import pallas as pl
from jax.experimental.pallas import tpu as pltpu
from jax.experimental.pallas import tpu_sc as plsc

D = 128
N1 = 10000
E = 320000
K1, K2, K3 = 5000, 2500, 1250

NW = 32
NS = 16
EPT = E // NW
EB = 80
NEB = EPT // EB

CH1, NP1 = 320, 32 * 320
CH2, NP2 = 192, 32 * 192
CH3, NP3 = 112, 32 * 112
CH4, NP4 = 40, 32 * 40

BR = 256
TRD = 1024
NEG_INF = float("-inf")

_mesh = functools.partial(
    plsc.VectorSubcoreMesh, core_axis_name="c", subcore_axis_name="s",
    num_cores=2, num_subcores=NS)


def _wid():
  return lax.axis_index("c") * NS + lax.axis_index("s")


def _splat(v, lane):
  idx = jnp.zeros((16,), jnp.int32) + lane
  return v.at[idx].get(mode="promise_in_bounds")


def _zero_1d(ref, nwords):
  z = jnp.zeros((16,), ref.dtype)
  def body(i, _):
    ref[pl.ds(i * 16, 16)] = z
    return 0
  lax.fori_loop(0, nwords // 16, body, 0)


def _sc_deg1(dst3):
  def body(dst_hbm, out_hbm, dbuf, degv):
    w = _wid()
    pltpu.sync_copy(dst_hbm.at[w], dbuf)
    _zero_1d(degv, NP1)
    ones = jnp.ones((16,), jnp.float32)
    def bstep(b, _):
      for j in range(EB // 16):
        d16 = dbuf[b, pl.ds(j * 16, 16)]
        plsc.addupdate_scatter(degv, [d16], ones)
      return 0
    lax.fori_loop(0, NEB, bstep, 0)
    pltpu.sync_copy(degv, out_hbm.at[w])

  return pl.kernel(
      body,
      out_type=jax.ShapeDtypeStruct((NW, NP1), jnp.float32),
      mesh=_mesh(),
      compiler_params=pltpu.CompilerParams(needs_layout_passes=False),
      scratch_types=[
          pltpu.VMEM((NEB, EB), jnp.int32),
          pltpu.VMEM((NP1,), jnp.float32),
      ],
  )(dst3)


def _sc_msg(h, src3, dst3, NP):
  ZB = 16
  RPT = NP // NS
  EBM = 25
  NBUF = 8
  CPB = 40
  NCH = EPT // (CPB * EBM)

  def body(h_hbm, src_hbm, dst_hbm, out_hbm, sbuf, dbuf, rows, zrows, accS,
           *sems):
    gsems = sems[:NBUF]
    ssems = sems[NBUF:]
    c = lax.axis_index("c")
    s = lax.axis_index("s")
    w = c * NS + s
    z = jnp.zeros((16,), jnp.float32)
    def zrow(i, _):
      r = i // 8
      l = i - r * 8
      zrows[r, pl.ds(l * 16, 16)] = z
      return 0
    lax.fori_loop(0, ZB * 8, zrow, 0)
    def zcp(j, _):
      pltpu.sync_copy(zrows, accS.at[pl.ds(s * RPT + j * ZB, ZB)])
      return 0
    lax.fori_loop(0, RPT // ZB, zcp, 0)
    plsc.subcore_barrier()
    def chunk(ci, _):
      pltpu.sync_copy(src_hbm.at[w, ci], sbuf)
      pltpu.sync_copy(dst_hbm.at[w, ci], dbuf)
      def rnd(r, _):
        for i in range(NBUF):
          j = r * NBUF + i
          @pl.when(r > 0)
          def _():
            pltpu.make_async_copy(rows.at[i], accS.at[dbuf.at[j]],
                                  ssems[i]).wait()
          pltpu.async_copy(h_hbm.at[sbuf.at[j]], rows.at[i], gsems[i])
        for i in range(NBUF):
          j = r * NBUF + i
          pltpu.make_async_copy(h_hbm.at[sbuf.at[j]], rows.at[i],
                                gsems[i]).wait()
          pltpu.async_copy(rows.at[i], accS.at[dbuf.at[j]], ssems[i],
                           add=True)
        return 0
      lax.fori_loop(0, CPB // NBUF, rnd, 0)
      for i in range(NBUF):
        pltpu.make_async_copy(rows.at[i], accS.at[dbuf.at[CPB - NBUF + i]],
                              ssems[i]).wait()
      return 0
    lax.fori_loop(0, NCH, chunk, 0)
    plsc.subcore_barrier()
    def ocp(j, _):
      pltpu.sync_copy(accS.at[pl.ds(s * RPT + j * ZB, ZB)],
                      out_hbm.at[c, pl.ds(s * RPT + j * ZB, ZB)])
      return 0
    lax.fori_loop(0, RPT // ZB, ocp, 0)

  return pl.kernel(
      body,
      out_type=jax.ShapeDtypeStruct((2, NP, D), jnp.float32),
      mesh=_mesh(),
      compiler_params=pltpu.CompilerParams(needs_layout_passes=False),
      scratch_types=[
          pltpu.VMEM((CPB, EBM), jnp.int32),
          pltpu.VMEM((CPB, EBM), jnp.int32),
          pltpu.VMEM((NBUF, EBM, D), jnp.float32),
          pltpu.VMEM((ZB, D), jnp.float32),
          pltpu.VMEM_SHARED((NP, D), jnp.float32),
      ] + [pltpu.SemaphoreType.DMA] * 16,
  )(h, src3.reshape(NW, NCH, CPB, EBM), dst3.reshape(NW, NCH, CPB, EBM))


def _sc_score(t, src3, dst3, NP):
  def body(t_hbm, src_hbm, dst_hbm, out_hbm, tv, sbuf, dbuf, scv):
    w = _wid()
    pltpu.sync_copy(t_hbm, tv)
    pltpu.sync_copy(src_hbm.at[w], sbuf)
    pltpu.sync_copy(dst_hbm.at[w], dbuf)
    _zero_1d(scv, NP)
    def bstep(b, _):
      for j in range(EB // 16):
        s16 = sbuf[b, pl.ds(j * 16, 16)]
        d16 = dbuf[b, pl.ds(j * 16, 16)]
        tv16 = plsc.load_gather(tv, [s16])
        plsc.addupdate_scatter(scv, [d16], tv16)
      return 0
    lax.fori_loop(0, NEB, bstep, 0)
    pltpu.sync_copy(scv, out_hbm.at[w])

  return pl.kernel(
      body,
      out_type=jax.ShapeDtypeStruct((NW, NP), jnp.float32),
      mesh=_mesh(),
      compiler_params=pltpu.CompilerParams(needs_layout_passes=False),
      scratch_types=[
          pltpu.VMEM((NP,), jnp.float32),
          pltpu.VMEM((NEB, EB), jnp.int32),
          pltpu.VMEM((NEB, EB), jnp.int32),
          pltpu.VMEM((NP,), jnp.float32),
      ],
  )(t, src3, dst3)


def _sc_select(skey, gate, params, g, CH, NPn, trash, SB):
  NBX = CH // SB
  SG = SB // 16

  def body(skey_hbm, gate_hbm, prm_hbm, g_hbm, snx_hbm, xnew_hbm,
           skv, gatev, prmv, rows, selv, nidx2d, sem):
    w = _wid()
    pltpu.sync_copy(skey_hbm.at[w], skv)
    pltpu.sync_copy(gate_hbm.at[w], gatev)
    pltpu.sync_copy(prm_hbm.at[w], prmv)
    pltpu.sync_copy(g_hbm.at[pl.ds(w * CH, CH)], rows)
    prm = prmv[...]
    eq_carry = _splat(prm, 0)
    pos_carry = _splat(prm, 1)
    rsp = _splat(prm, 2)
    tkey = _splat(prm, 3)
    trash_v = jnp.zeros((16,), jnp.int32) + trash
    neg1 = jnp.zeros((16,), jnp.int32) - 1
    last = jnp.zeros((16,), jnp.int32) + 15
    for v in range(CH // 16):
      sk = skv[pl.ds(v * 16, 16)]
      gt = sk > tkey
      eqm = sk == tkey
      eqi = eqm.astype(jnp.int32)
      cse = plsc.cumsum(eqi)
      rank = eq_carry + (cse - eqi)
      sel16 = gt | (eqm & (rank < rsp))
      seli = sel16.astype(jnp.int32)
      css = plsc.cumsum(seli)
      npos = pos_carry + (css - seli)
      nidx2d[v // SG, pl.ds((v % SG) * 16, 16)] = jnp.where(
          sel16, npos, trash_v)
      selv[pl.ds(v * 16, 16)] = jnp.where(sel16, npos, neg1)
      eq_carry = eq_carry + cse.at[last].get(mode="promise_in_bounds")
      pos_carry = pos_carry + css.at[last].get(mode="promise_in_bounds")
    def rscale(r, _):
      v = r // 16
      lane = r - v * 16
      gsp = _splat(gatev[pl.ds(v * 16, 16)], lane)
      for f in range(D // 16):
        rows[r, pl.ds(f * 16, 16)] = rows[r, pl.ds(f * 16, 16)] * gsp
      return 0
    lax.fori_loop(0, CH, rscale, 0)
    for bx in range(NBX):
      pltpu.async_copy(rows.at[pl.ds(bx * SB, SB)],
                       xnew_hbm.at[nidx2d.at[bx]], sem).wait()
    pltpu.sync_copy(selv, snx_hbm.at[w])

  return pl.kernel(
      body,
      out_type=(
          jax.ShapeDtypeStruct((NW, CH), jnp.int32),
          jax.ShapeDtypeStruct((NPn, D), jnp.float32),
      ),
      mesh=_mesh(),
      compiler_params=pltpu.CompilerParams(needs_layout_passes=False),
      scratch_types=[
          pltpu.VMEM((CH,), jnp.int32),
          pltpu.VMEM((CH,), jnp.float32),
          pltpu.VMEM((16,), jnp.int32),
          pltpu.VMEM((CH, D), jnp.float32),
          pltpu.VMEM((CH,), jnp.int32),
          pltpu.VMEM((NBX, SB), jnp.int32),
          pltpu.SemaphoreType.DMA,
      ],
  )(skey, gate, params, g)


def _sc_reindex(src3, dst3, snx, NP, NPn, trash):
  def body(src_hbm, dst_hbm, snx_hbm, srcn_hbm, dstn_hbm, deg_hbm,
           snxv, sbuf, dbuf, sobuf, dobuf, degv):
    w = _wid()
    pltpu.sync_copy(snx_hbm, snxv)
    pltpu.sync_copy(src_hbm.at[w], sbuf)
    pltpu.sync_copy(dst_hbm.at[w], dbuf)
    _zero_1d(degv, NPn)
    ones = jnp.ones((16,), jnp.float32)
    trash_v = jnp.zeros((16,), jnp.int32) + trash
    iota16 = lax.iota(jnp.int32, 16)
    def bstep(b, _):
      for j in range(EB // 16):
        s16 = sbuf[b, pl.ds(j * 16, 16)]
        d16 = dbuf[b, pl.ds(j * 16, 16)]
        gs = plsc.load_gather(snxv, [s16])
        gd = plsc.load_gather(snxv, [d16])
        valid = (gs >= 0) & (gd >= 0)
        spread = trash_v + ((b * EB + j * 16 + iota16) & (TRD - 1))
        nd = jnp.where(valid, gd, spread)
        sobuf[b, pl.ds(j * 16, 16)] = jnp.where(valid, gs, trash_v)
        dobuf[b, pl.ds(j * 16, 16)] = nd
        plsc.addupdate_scatter(degv, [nd], ones)
      return 0
    lax.fori_loop(0, NEB, bstep, 0)
    pltpu.sync_copy(sobuf, srcn_hbm.at[w])
    pltpu.sync_copy(dobuf, dstn_hbm.at[w])
    pltpu.sync_copy(degv, deg_hbm.at[w])

  return pl.kernel(
      body,
      out_type=(
          jax.ShapeDtypeStruct((NW, NEB, EB), jnp.int32),
          jax.ShapeDtypeStruct((NW, NEB, EB), jnp.int32),
          jax.ShapeDtypeStruct((NW, NPn), jnp.float32),
      ),
      mesh=_mesh(),
      compiler_params=pltpu.CompilerParams(needs_layout_passes=False),
      scratch_types=[
          pltpu.VMEM((NP,), jnp.int32),
          pltpu.VMEM((NEB, EB), jnp.int32),
          pltpu.VMEM((NEB, EB), jnp.int32),
          pltpu.VMEM((NEB, EB), jnp.int32),
          pltpu.VMEM((NEB, EB), jnp.int32),
          pltpu.VMEM((NPn,), jnp.float32),
      ],
  )(src3, dst3, snx)


def _tc_pre(x, W, deg_part, n, NP, with_readout):
  RB = NP // BR

  def body(*refs):
    if with_readout:
      x_ref, w_ref, dp_ref, h_ref, r_ref = refs
    else:
      x_ref, w_ref, dp_ref, h_ref = refs
    i = pl.program_id(0)
    deg = jnp.sum(dp_ref[...], axis=0) + 1.0
    dinv = lax.rsqrt(deg)
    rowid = i * BR + lax.broadcasted_iota(jnp.int32, (BR, D), 0)
    mask = rowid < n
    xb = x_ref[...]
    h = jnp.dot(xb, w_ref[...], preferred_element_type=jnp.float32)
    h_ref[...] = jnp.where(mask, h * dinv[:, None], 0.0)
    if with_readout:
      @pl.when(i == 0)
      def _():
        r_ref[...] = jnp.concatenate(
            [jnp.zeros((1, D), jnp.float32),
             jnp.full((1, D), NEG_INF, jnp.float32)], axis=1)
      sm = jnp.sum(jnp.where(mask, xb, 0.0), axis=0, keepdims=True) / n
      mx = jnp.max(jnp.where(mask, xb, NEG_INF), axis=0, keepdims=True)
      r_ref[:, 0:D] = r_ref[:, 0:D] + sm
      r_ref[:, D:2 * D] = jnp.maximum(r_ref[:, D:2 * D], mx)

  out_shape = [jax.ShapeDtypeStruct((NP, D), jnp.float32)]
  out_specs = [pl.BlockSpec((BR, D), lambda i: (i, 0))]
  if with_readout:
    out_shape.append(jax.ShapeDtypeStruct((1, 2 * D), jnp.float32))
    out_specs.append(pl.BlockSpec((1, 2 * D), lambda i: (0, 0)))
  res = pl.pallas_call(
      body,
      grid=(RB,),
      in_specs=[
          pl.BlockSpec((BR, D), lambda i: (i, 0)),
          pl.BlockSpec((D, D), lambda i: (0, 0)),
          pl.BlockSpec((NW, BR), lambda i: (0, i)),
      ],
      out_specs=out_specs,
      out_shape=out_shape,
  )(x, W, deg_part)
  return res if with_readout else res[0]


def _tc_mid(acc2, h, deg_part, b, wru, n, NP):
  RB = NP // BR

  def body(a_ref, h_ref, dp_ref, b_ref, w_ref, g_ref, tu_ref):
    i = pl.program_id(0)
    deg = jnp.sum(dp_ref[...], axis=0) + 1.0
    dinv = lax.rsqrt(deg)
    rowid = i * BR + lax.broadcasted_iota(jnp.int32, (BR, D), 0)
    mask = rowid < n
    acc = a_ref[0] + a_ref[1] + h_ref[...]
    g = jnp.maximum(acc * dinv[:, None] + b_ref[...], 0.0)
    g = jnp.where(mask, g, 0.0)
    g_ref[...] = g
    tu_ref[...] = jnp.dot(g, w_ref[...], preferred_element_type=jnp.float32)

  return pl.pallas_call(
      body,
      grid=(RB,),
      in_specs=[
          pl.BlockSpec((2, BR, D), lambda i: (0, i, 0)),
          pl.BlockSpec((BR, D), lambda i: (i, 0)),
          pl.BlockSpec((NW, BR), lambda i: (0, i)),
          pl.BlockSpec((1, D), lambda i: (0, 0)),
          pl.BlockSpec((D, 2), lambda i: (0, 0)),
      ],
      out_specs=[
          pl.BlockSpec((BR, D), lambda i: (i, 0)),
          pl.BlockSpec((BR, 2), lambda i: (i, 0)),
      ],
      out_shape=[
          jax.ShapeDtypeStruct((NP, D), jnp.float32),
          jax.ShapeDtypeStruct((NP, 2), jnp.float32),
      ],
  )(acc2, h, deg_part, b, wru)


def _tc_selprep(scp, u2d, brel, n, k, CH):
  def body(scp_ref, u_ref, br_ref, score_ref, skey_ref, gate_ref, prm_ref):
    INT_MIN = jnp.int32(-2147483648)
    a = pl.program_id(0)

    @pl.when(a == 0)
    def _():
      score_ref[...] = jnp.zeros((NW, CH), jnp.float32)
    score_ref[...] = score_ref[...] + scp_ref[0]

    @pl.when(a == NW - 1)
    def _():
      s = score_ref[...] + u_ref[...] + br_ref[0, 0]
      flat = (lax.broadcasted_iota(jnp.int32, (NW, CH), 0) * CH
              + lax.broadcasted_iota(jnp.int32, (NW, CH), 1))
      s = jnp.where(flat < n, s, NEG_INF)
      score_ref[...] = s
      bits = lax.bitcast_convert_type(s, jnp.int32)
      m = lax.shift_right_arithmetic(bits, 31)
      skey = bits ^ (m & jnp.int32(0x7FFFFFFF))
      skey_ref[...] = skey
      kf = jnp.float32(k)

      def bstep(i, tu):
        bit = 31 - i
        cand = tu | lax.shift_left(jnp.int32(1), bit)
        thr = cand ^ INT_MIN
        c = jnp.sum((skey >= thr).astype(jnp.float32))
        return jnp.where(c >= kf, cand, tu)

      tu = lax.fori_loop(0, 32, bstep, jnp.int32(0))
      tkey = tu ^ INT_MIN
      cgt = jnp.sum((skey > tkey).astype(jnp.float32))
      r = kf - cgt
      gt_t = jnp.sum((skey > tkey).astype(jnp.float32), axis=1, keepdims=True)
      eq_t = jnp.sum((skey == tkey).astype(jnp.float32), axis=1, keepdims=True)
      tri = (lax.broadcasted_iota(jnp.int32, (NW, NW), 0)
             > lax.broadcasted_iota(jnp.int32, (NW, NW), 1)).astype(jnp.float32)
      gt_b = jnp.dot(tri, gt_t, preferred_element_type=jnp.float32)
      eq_b = jnp.dot(tri, eq_t, preferred_element_type=jnp.float32)
      base = gt_b + jnp.minimum(eq_b, r)
      prm = jnp.concatenate([
          eq_b.astype(jnp.int32),
          base.astype(jnp.int32),
          jnp.full((NW, 1), r, jnp.float32).astype(jnp.int32),
          jnp.full((NW, 1), tkey, jnp.int32),
          jnp.zeros((NW, 12), jnp.int32),
      ], axis=1)
      prm_ref[...] = prm
      gate_ref[...] = jnp.tanh(s)

  return pl.pallas_call(
      body,
      grid=(NW,),
      in_specs=[
          pl.BlockSpec((1, NW, CH), lambda a: (a, 0, 0)),
          pl.BlockSpec((NW, CH), lambda a: (0, 0)),
          pl.BlockSpec((1, 1), lambda a: (0, 0)),
      ],
      out_specs=[
          pl.BlockSpec((NW, CH), lambda a: (0, 0)),
          pl.BlockSpec((NW, CH), lambda a: (0, 0)),
          pl.BlockSpec((NW, CH), lambda a: (0, 0)),
          pl.BlockSpec((NW, 16), lambda a: (0, 0)),
      ],
      out_shape=[
          jax.ShapeDtypeStruct((NW, CH), jnp.float32),
          jax.ShapeDtypeStruct((NW, CH), jnp.int32),
          jax.ShapeDtypeStruct((NW, CH), jnp.float32),
          jax.ShapeDtypeStruct((NW, 16), jnp.int32),
      ],
  )(scp, u2d, brel)


def _tc_head(xnew3, r1, r2, M1, bm1, M2, bm2, M3, bm3):
  def body(x_ref, r1_ref, r2_ref, m1_ref, b1_ref, m2_ref, b2_ref, m3_ref,
           b3_ref, o_ref):
    mask = lax.broadcasted_iota(jnp.int32, (NP4, D), 0) < K3
    xb = x_ref[...]
    sm = jnp.sum(jnp.where(mask, xb, 0.0), axis=0, keepdims=True) / K3
    mx = jnp.max(jnp.where(mask, xb, NEG_INF), axis=0, keepdims=True)
    r = r1_ref[...] + r2_ref[...] + jnp.concatenate([sm, mx], axis=1)
    h = jnp.maximum(
        jnp.dot(r, m1_ref[...], preferred_element_type=jnp.float32)
        + b1_ref[...], 0.0)
    h = jnp.maximum(
        jnp.dot(h, m2_ref[...], preferred_element_type=jnp.float32)
        + b2_ref[...], 0.0)
    y = (jnp.dot(h, m3_ref[...], preferred_element_type=jnp.float32)
         + b3_ref[...])
    ymax = jnp.max(y, axis=1, keepdims=True)
    e = jnp.exp(y - ymax)
    lse = jnp.log(jnp.sum(e, axis=1, keepdims=True))
    o_ref[...] = y - ymax - lse

  return pl.pallas_call(
      body,
      out_shape=jax.ShapeDtypeStruct((1, 10), jnp.float32),
  )(xnew3, r1, r2, M1, bm1, M2, bm2, M3, bm3)


def _stage_pool(g, tu, scp, brel, src3, dst3, n, k, CH, NP, NPn, last, SB):
  u2d = tu[:, 1].reshape(NW, CH)
  score, skey, gate, prm = _tc_selprep(
      scp.reshape(NW, NW, CH), u2d, brel.reshape(1, 1), n, k, CH)
  del score
  snx, xnew = _sc_select(skey, gate, prm, g, CH, NPn, k, SB)
  if last:
    return xnew, None, None, None
  src3n, dst3n, degn = _sc_reindex(src3, dst3, snx.reshape(NP), NP, NPn, k)
  return xnew, src3n, dst3n, degn


def kernel(x, edge_index, W1, b1, Wrel1, brel1, Wroot1, W2, b2, Wrel2, brel2,
           Wroot2, W3, b3, Wrel3, brel3, Wroot3, M1, bm1, M2, bm2, M3, bm3):
  src3 = edge_index[0].astype(jnp.int32).reshape(NW, NEB, EB)
  dst3 = edge_index[1].astype(jnp.int32).reshape(NW, NEB, EB)
  xp = jnp.pad(x, ((0, NP1 - N1), (0, 0)))

  wru1 = jnp.concatenate([Wrel1, Wroot1], axis=1)
  wru2 = jnp.concatenate([Wrel2, Wroot2], axis=1)
  wru3 = jnp.concatenate([Wrel3, Wroot3], axis=1)

  deg1 = _sc_deg1(dst3)
  h1 = _tc_pre(xp, W1, deg1, N1, NP1, False)
  acc1 = _sc_msg(h1, src3, dst3, NP1)
  g1, tu1 = _tc_mid(acc1, h1, deg1, b1.reshape(1, D), wru1, N1, NP1)
  scp1 = _sc_score(tu1[:, 0], src3, dst3, NP1)
  xnew1, src3b, dst3b, deg2 = _stage_pool(
      g1, tu1, scp1, brel1, src3, dst3, N1, K1, CH1, NP1, NP2, False, 80)

  h2, r1 = _tc_pre(xnew1, W2, deg2, K1, NP2, True)
  acc2 = _sc_msg(h2, src3b, dst3b, NP2)
  g2, tu2 = _tc_mid(acc2, h2, deg2, b2.reshape(1, D), wru2, K1, NP2)
  scp2 = _sc_score(tu2[:, 0], src3b, dst3b, NP2)
  xnew2, src3c, dst3c, deg3 = _stage_pool(
      g2, tu2, scp2, brel2, src3b, dst3b, K1, K2, CH2, NP2, NP3, False, 96)

  h3, r2 = _tc_pre(xnew2, W3, deg3, K2, NP3, True)
  acc3 = _sc_msg(h3, src3c, dst3c, NP3)
  g3, tu3 = _tc_mid(acc3, h3, deg3, b3.reshape(1, D), wru3, K2, NP3)
  scp3 = _sc_score(tu3[:, 0], src3c, dst3c, NP3)
  xnew3, _, _, _ = _stage_pool(
      g3, tu3, scp3, brel3, src3c, dst3c, K2, K3, CH3, NP3, NP4, True, 112)

  return _tc_head(xnew3, r1, r2, M1, bm1.reshape(1, D), M2,
                  bm2.reshape(1, D // 2), M3, bm3.reshape(1, 10))


# speedup vs baseline: 17.4698x; 17.4698x over previous
import functools

import jax
import jax.numpy as jnp
from jax import lax
from jax.experimental import pallas as pl
from jax.experimental.pallas import tpu as pltpu
from jax.experimental.pallas import tpu_sc as plsc

D = 128
N1 = 10000
E = 320000
K1, K2, K3 = 5000, 2500, 1250

NW = 32
NS = 16
EPT = E // NW
EB = 80
NEB = EPT // EB

CH1, NP1 = 320, 32 * 320
CH2, NP2 = 192, 32 * 192
CH3, NP3 = 112, 32 * 112
CH4, NP4 = 40, 32 * 40

BR = 256
TRD = 1024
NEG_INF = float("-inf")

_mesh = functools.partial(
    plsc.VectorSubcoreMesh, core_axis_name="c", subcore_axis_name="s",
    num_cores=2, num_subcores=NS)


def _wid():
  return lax.axis_index("c") * NS + lax.axis_index("s")


def _splat(v, lane):
  idx = jnp.zeros((16,), jnp.int32) + lane
  return v.at[idx].get(mode="promise_in_bounds")


def _zero_1d(ref, nwords):
  z = jnp.zeros((16,), ref.dtype)
  def body(i, _):
    ref[pl.ds(i * 16, 16)] = z
    return 0
  lax.fori_loop(0, nwords // 16, body, 0)


def _sc_deg1(dst3):
  def body(dst_hbm, out_hbm, dbuf, degv):
    w = _wid()
    pltpu.sync_copy(dst_hbm.at[w], dbuf)
    _zero_1d(degv, NP1)
    ones = jnp.ones((16,), jnp.float32)
    def bstep(b, _):
      for j in range(EB // 16):
        d16 = dbuf[b, pl.ds(j * 16, 16)]
        plsc.addupdate_scatter(degv, [d16], ones)
      return 0
    lax.fori_loop(0, NEB, bstep, 0)
    pltpu.sync_copy(degv, out_hbm.at[w])

  return pl.kernel(
      body,
      out_type=jax.ShapeDtypeStruct((NW, NP1), jnp.float32),
      mesh=_mesh(),
      compiler_params=pltpu.CompilerParams(needs_layout_passes=False),
      scratch_types=[
          pltpu.VMEM((NEB, EB), jnp.int32),
          pltpu.VMEM((NP1,), jnp.float32),
      ],
  )(dst3)


def _sc_msg(h, src3, dst3, NP):
  ZB = 16
  RPT = NP // NS
  EBM = 25
  NBUF = 8
  CPB = 40
  NCH = EPT // (CPB * EBM)

  def body(h_hbm, src_hbm, dst_hbm, out_hbm, sbuf, dbuf, rows, zrows, accS,
           *sems):
    gsems = sems[:NBUF]
    ssems = sems[NBUF:]
    c = lax.axis_index("c")
    s = lax.axis_index("s")
    w = c * NS + s
    z = jnp.zeros((16,), jnp.float32)
    def zrow(i, _):
      r = i // 8
      l = i - r * 8
      zrows[r, pl.ds(l * 16, 16)] = z
      return 0
    lax.fori_loop(0, ZB * 8, zrow, 0)
    def zcp(j, _):
      pltpu.sync_copy(zrows, accS.at[pl.ds(s * RPT + j * ZB, ZB)])
      return 0
    lax.fori_loop(0, RPT // ZB, zcp, 0)
    plsc.subcore_barrier()
    def chunk(ci, _):
      pltpu.sync_copy(src_hbm.at[w, ci], sbuf)
      pltpu.sync_copy(dst_hbm.at[w, ci], dbuf)
      def rnd(r, _):
        for i in range(NBUF):
          j = r * NBUF + i
          @pl.when(r > 0)
          def _():
            pltpu.make_async_copy(rows.at[i], accS.at[dbuf.at[j]],
                                  ssems[i]).wait()
          pltpu.async_copy(h_hbm.at[sbuf.at[j]], rows.at[i], gsems[i])
        for i in range(NBUF):
          j = r * NBUF + i
          pltpu.make_async_copy(h_hbm.at[sbuf.at[j]], rows.at[i],
                                gsems[i]).wait()
          pltpu.async_copy(rows.at[i], accS.at[dbuf.at[j]], ssems[i],
                           add=True)
        return 0
      lax.fori_loop(0, CPB // NBUF, rnd, 0)
      for i in range(NBUF):
        pltpu.make_async_copy(rows.at[i], accS.at[dbuf.at[CPB - NBUF + i]],
                              ssems[i]).wait()
      return 0
    lax.fori_loop(0, NCH, chunk, 0)
    plsc.subcore_barrier()
    def ocp(j, _):
      pltpu.sync_copy(accS.at[pl.ds(s * RPT + j * ZB, ZB)],
                      out_hbm.at[c, pl.ds(s * RPT + j * ZB, ZB)])
      return 0
    lax.fori_loop(0, RPT // ZB, ocp, 0)

  return pl.kernel(
      body,
      out_type=jax.ShapeDtypeStruct((2, NP, D), jnp.float32),
      mesh=_mesh(),
      compiler_params=pltpu.CompilerParams(needs_layout_passes=False),
      scratch_types=[
          pltpu.VMEM((CPB, EBM), jnp.int32),
          pltpu.VMEM((CPB, EBM), jnp.int32),
          pltpu.VMEM((NBUF, EBM, D), jnp.float32),
          pltpu.VMEM((ZB, D), jnp.float32),
          pltpu.VMEM_SHARED((NP, D), jnp.float32),
      ] + [pltpu.SemaphoreType.DMA] * 16,
  )(h, src3.reshape(NW, NCH, CPB, EBM), dst3.reshape(NW, NCH, CPB, EBM))


def _sc_score(t, src3, dst3, NP):
  def body(t_hbm, src_hbm, dst_hbm, out_hbm, tv, sbuf, dbuf, scv):
    w = _wid()
    pltpu.sync_copy(t_hbm, tv)
    pltpu.sync_copy(src_hbm.at[w], sbuf)
    pltpu.sync_copy(dst_hbm.at[w], dbuf)
    _zero_1d(scv, NP)
    def bstep(b, _):
      for j in range(EB // 16):
        s16 = sbuf[b, pl.ds(j * 16, 16)]
        d16 = dbuf[b, pl.ds(j * 16, 16)]
        tv16 = plsc.load_gather(tv, [s16])
        plsc.addupdate_scatter(scv, [d16], tv16)
      return 0
    lax.fori_loop(0, NEB, bstep, 0)
    pltpu.sync_copy(scv, out_hbm.at[w])

  return pl.kernel(
      body,
      out_type=jax.ShapeDtypeStruct((NW, NP), jnp.float32),
      mesh=_mesh(),
      compiler_params=pltpu.CompilerParams(needs_layout_passes=False),
      scratch_types=[
          pltpu.VMEM((NP,), jnp.float32),
          pltpu.VMEM((NEB, EB), jnp.int32),
          pltpu.VMEM((NEB, EB), jnp.int32),
          pltpu.VMEM((NP,), jnp.float32),
      ],
  )(t, src3, dst3)


def _sc_select(skey, gate, params, g, CH, NPn, trash, SB):
  NBX = CH // SB
  SG = SB // 16

  def body(skey_hbm, gate_hbm, prm_hbm, g_hbm, snx_hbm, xnew_hbm,
           skv, gatev, prmv, rows, selv, nidx2d, sem):
    w = _wid()
    pltpu.sync_copy(skey_hbm.at[w], skv)
    pltpu.sync_copy(gate_hbm.at[w], gatev)
    pltpu.sync_copy(prm_hbm.at[w], prmv)
    pltpu.sync_copy(g_hbm.at[pl.ds(w * CH, CH)], rows)
    prm = prmv[...]
    eq_carry = _splat(prm, 0)
    pos_carry = _splat(prm, 1)
    rsp = _splat(prm, 2)
    tkey = _splat(prm, 3)
    trash_v = jnp.zeros((16,), jnp.int32) + trash
    neg1 = jnp.zeros((16,), jnp.int32) - 1
    last = jnp.zeros((16,), jnp.int32) + 15
    for v in range(CH // 16):
      sk = skv[pl.ds(v * 16, 16)]
      gt = sk > tkey
      eqm = sk == tkey
      eqi = eqm.astype(jnp.int32)
      cse = plsc.cumsum(eqi)
      rank = eq_carry + (cse - eqi)
      sel16 = gt | (eqm & (rank < rsp))
      seli = sel16.astype(jnp.int32)
      css = plsc.cumsum(seli)
      npos = pos_carry + (css - seli)
      nidx2d[v // SG, pl.ds((v % SG) * 16, 16)] = jnp.where(
          sel16, npos, trash_v)
      selv[pl.ds(v * 16, 16)] = jnp.where(sel16, npos, neg1)
      eq_carry = eq_carry + cse.at[last].get(mode="promise_in_bounds")
      pos_carry = pos_carry + css.at[last].get(mode="promise_in_bounds")
    def rscale(r, _):
      v = r // 16
      lane = r - v * 16
      gsp = _splat(gatev[pl.ds(v * 16, 16)], lane)
      for f in range(D // 16):
        rows[r, pl.ds(f * 16, 16)] = rows[r, pl.ds(f * 16, 16)] * gsp
      return 0
    lax.fori_loop(0, CH, rscale, 0)
    for bx in range(NBX):
      pltpu.async_copy(rows.at[pl.ds(bx * SB, SB)],
                       xnew_hbm.at[nidx2d.at[bx]], sem).wait()
    pltpu.sync_copy(selv, snx_hbm.at[w])

  return pl.kernel(
      body,
      out_type=(
          jax.ShapeDtypeStruct((NW, CH), jnp.int32),
          jax.ShapeDtypeStruct((NPn, D), jnp.float32),
      ),
      mesh=_mesh(),
      compiler_params=pltpu.CompilerParams(needs_layout_passes=False),
      scratch_types=[
          pltpu.VMEM((CH,), jnp.int32),
          pltpu.VMEM((CH,), jnp.float32),
          pltpu.VMEM((16,), jnp.int32),
          pltpu.VMEM((CH, D), jnp.float32),
          pltpu.VMEM((CH,), jnp.int32),
          pltpu.VMEM((NBX, SB), jnp.int32),
          pltpu.SemaphoreType.DMA,
      ],
  )(skey, gate, params, g)


def _sc_reindex(src3, dst3, snx, NP, NPn, trash):
  def body(src_hbm, dst_hbm, snx_hbm, srcn_hbm, dstn_hbm, deg_hbm,
           snxv, sbuf, dbuf, sobuf, dobuf, degv):
    w = _wid()
    pltpu.sync_copy(snx_hbm, snxv)
    pltpu.sync_copy(src_hbm.at[w], sbuf)
    pltpu.sync_copy(dst_hbm.at[w], dbuf)
    _zero_1d(degv, NPn)
    ones = jnp.ones((16,), jnp.float32)
    trash_v = jnp.zeros((16,), jnp.int32) + trash
    iota16 = lax.iota(jnp.int32, 16)
    def bstep(b, _):
      for j in range(EB // 16):
        s16 = sbuf[b, pl.ds(j * 16, 16)]
        d16 = dbuf[b, pl.ds(j * 16, 16)]
        gs = plsc.load_gather(snxv, [s16])
        gd = plsc.load_gather(snxv, [d16])
        valid = (gs >= 0) & (gd >= 0)
        spread = trash_v + ((b * EB + j * 16 + iota16) & (TRD - 1))
        nd = jnp.where(valid, gd, spread)
        sobuf[b, pl.ds(j * 16, 16)] = jnp.where(valid, gs, spread)
        dobuf[b, pl.ds(j * 16, 16)] = nd
        plsc.addupdate_scatter(degv, [nd], ones)
      return 0
    lax.fori_loop(0, NEB, bstep, 0)
    pltpu.sync_copy(sobuf, srcn_hbm.at[w])
    pltpu.sync_copy(dobuf, dstn_hbm.at[w])
    pltpu.sync_copy(degv, deg_hbm.at[w])

  return pl.kernel(
      body,
      out_type=(
          jax.ShapeDtypeStruct((NW, NEB, EB), jnp.int32),
          jax.ShapeDtypeStruct((NW, NEB, EB), jnp.int32),
          jax.ShapeDtypeStruct((NW, NPn), jnp.float32),
      ),
      mesh=_mesh(),
      compiler_params=pltpu.CompilerParams(needs_layout_passes=False),
      scratch_types=[
          pltpu.VMEM((NP,), jnp.int32),
          pltpu.VMEM((NEB, EB), jnp.int32),
          pltpu.VMEM((NEB, EB), jnp.int32),
          pltpu.VMEM((NEB, EB), jnp.int32),
          pltpu.VMEM((NEB, EB), jnp.int32),
          pltpu.VMEM((NPn,), jnp.float32),
      ],
  )(src3, dst3, snx)


def _tc_pre(x, W, deg_part, n, NP, with_readout):
  RB = NP // BR

  def body(*refs):
    if with_readout:
      x_ref, w_ref, dp_ref, h_ref, r_ref = refs
    else:
      x_ref, w_ref, dp_ref, h_ref = refs
    i = pl.program_id(0)
    deg = jnp.sum(dp_ref[...], axis=0) + 1.0
    dinv = lax.rsqrt(deg)
    rowid = i * BR + lax.broadcasted_iota(jnp.int32, (BR, D), 0)
    mask = rowid < n
    xb = x_ref[...]
    h = jnp.dot(xb, w_ref[...], preferred_element_type=jnp.float32)
    h_ref[...] = jnp.where(mask, h * dinv[:, None], 0.0)
    if with_readout:
      @pl.when(i == 0)
      def _():
        r_ref[...] = jnp.concatenate(
            [jnp.zeros((1, D), jnp.float32),
             jnp.full((1, D), NEG_INF, jnp.float32)], axis=1)
      sm = jnp.sum(jnp.where(mask, xb, 0.0), axis=0, keepdims=True) / n
      mx = jnp.max(jnp.where(mask, xb, NEG_INF), axis=0, keepdims=True)
      r_ref[:, 0:D] = r_ref[:, 0:D] + sm
      r_ref[:, D:2 * D] = jnp.maximum(r_ref[:, D:2 * D], mx)

  out_shape = [jax.ShapeDtypeStruct((NP, D), jnp.float32)]
  out_specs = [pl.BlockSpec((BR, D), lambda i: (i, 0))]
  if with_readout:
    out_shape.append(jax.ShapeDtypeStruct((1, 2 * D), jnp.float32))
    out_specs.append(pl.BlockSpec((1, 2 * D), lambda i: (0, 0)))
  res = pl.pallas_call(
      body,
      grid=(RB,),
      in_specs=[
          pl.BlockSpec((BR, D), lambda i: (i, 0)),
          pl.BlockSpec((D, D), lambda i: (0, 0)),
          pl.BlockSpec((NW, BR), lambda i: (0, i)),
      ],
      out_specs=out_specs,
      out_shape=out_shape,
  )(x, W, deg_part)
  return res if with_readout else res[0]


def _tc_mid(acc2, h, deg_part, b, wru, n, NP):
  RB = NP // BR

  def body(a_ref, h_ref, dp_ref, b_ref, w_ref, g_ref, tu_ref):
    i = pl.program_id(0)
    deg = jnp.sum(dp_ref[...], axis=0) + 1.0
    dinv = lax.rsqrt(deg)
    rowid = i * BR + lax.broadcasted_iota(jnp.int32, (BR, D), 0)
    mask = rowid < n
    acc = a_ref[0] + a_ref[1] + h_ref[...]
    g = jnp.maximum(acc * dinv[:, None] + b_ref[...], 0.0)
    g = jnp.where(mask, g, 0.0)
    g_ref[...] = g
    tu_ref[...] = jnp.dot(g, w_ref[...], preferred_element_type=jnp.float32)

  return pl.pallas_call(
      body,
      grid=(RB,),
      in_specs=[
          pl.BlockSpec((2, BR, D), lambda i: (0, i, 0)),
          pl.BlockSpec((BR, D), lambda i: (i, 0)),
          pl.BlockSpec((NW, BR), lambda i: (0, i)),
          pl.BlockSpec((1, D), lambda i: (0, 0)),
          pl.BlockSpec((D, 2), lambda i: (0, 0)),
      ],
      out_specs=[
          pl.BlockSpec((BR, D), lambda i: (i, 0)),
          pl.BlockSpec((BR, 2), lambda i: (i, 0)),
      ],
      out_shape=[
          jax.ShapeDtypeStruct((NP, D), jnp.float32),
          jax.ShapeDtypeStruct((NP, 2), jnp.float32),
      ],
  )(acc2, h, deg_part, b, wru)


def _tc_selprep(scp, u2d, brel, n, k, CH):
  def body(scp_ref, u_ref, br_ref, score_ref, skey_ref, gate_ref, prm_ref):
    INT_MIN = jnp.int32(-2147483648)
    a = pl.program_id(0)

    @pl.when(a == 0)
    def _():
      score_ref[...] = jnp.zeros((NW, CH), jnp.float32)
    score_ref[...] = score_ref[...] + scp_ref[0]

    @pl.when(a == NW - 1)
    def _():
      s = score_ref[...] + u_ref[...] + br_ref[0, 0]
      flat = (lax.broadcasted_iota(jnp.int32, (NW, CH), 0) * CH
              + lax.broadcasted_iota(jnp.int32, (NW, CH), 1))
      s = jnp.where(flat < n, s, NEG_INF)
      score_ref[...] = s
      bits = lax.bitcast_convert_type(s, jnp.int32)
      m = lax.shift_right_arithmetic(bits, 31)
      skey = bits ^ (m & jnp.int32(0x7FFFFFFF))
      skey_ref[...] = skey
      kf = jnp.float32(k)

      def bstep(i, tu):
        bit = 31 - i
        cand = tu | lax.shift_left(jnp.int32(1), bit)
        thr = cand ^ INT_MIN
        c = jnp.sum((skey >= thr).astype(jnp.float32))
        return jnp.where(c >= kf, cand, tu)

      tu = lax.fori_loop(0, 32, bstep, jnp.int32(0))
      tkey = tu ^ INT_MIN
      cgt = jnp.sum((skey > tkey).astype(jnp.float32))
      r = kf - cgt
      gt_t = jnp.sum((skey > tkey).astype(jnp.float32), axis=1, keepdims=True)
      eq_t = jnp.sum((skey == tkey).astype(jnp.float32), axis=1, keepdims=True)
      tri = (lax.broadcasted_iota(jnp.int32, (NW, NW), 0)
             > lax.broadcasted_iota(jnp.int32, (NW, NW), 1)).astype(jnp.float32)
      gt_b = jnp.dot(tri, gt_t, preferred_element_type=jnp.float32)
      eq_b = jnp.dot(tri, eq_t, preferred_element_type=jnp.float32)
      base = gt_b + jnp.minimum(eq_b, r)
      prm = jnp.concatenate([
          eq_b.astype(jnp.int32),
          base.astype(jnp.int32),
          jnp.full((NW, 1), r, jnp.float32).astype(jnp.int32),
          jnp.full((NW, 1), tkey, jnp.int32),
          jnp.zeros((NW, 12), jnp.int32),
      ], axis=1)
      prm_ref[...] = prm
      gate_ref[...] = jnp.tanh(s)

  return pl.pallas_call(
      body,
      grid=(NW,),
      in_specs=[
          pl.BlockSpec((1, NW, CH), lambda a: (a, 0, 0)),
          pl.BlockSpec((NW, CH), lambda a: (0, 0)),
          pl.BlockSpec((1, 1), lambda a: (0, 0)),
      ],
      out_specs=[
          pl.BlockSpec((NW, CH), lambda a: (0, 0)),
          pl.BlockSpec((NW, CH), lambda a: (0, 0)),
          pl.BlockSpec((NW, CH), lambda a: (0, 0)),
          pl.BlockSpec((NW, 16), lambda a: (0, 0)),
      ],
      out_shape=[
          jax.ShapeDtypeStruct((NW, CH), jnp.float32),
          jax.ShapeDtypeStruct((NW, CH), jnp.int32),
          jax.ShapeDtypeStruct((NW, CH), jnp.float32),
          jax.ShapeDtypeStruct((NW, 16), jnp.int32),
      ],
  )(scp, u2d, brel)


def _tc_head(xnew3, r1, r2, M1, bm1, M2, bm2, M3, bm3):
  def body(x_ref, r1_ref, r2_ref, m1_ref, b1_ref, m2_ref, b2_ref, m3_ref,
           b3_ref, o_ref):
    mask = lax.broadcasted_iota(jnp.int32, (NP4, D), 0) < K3
    xb = x_ref[...]
    sm = jnp.sum(jnp.where(mask, xb, 0.0), axis=0, keepdims=True) / K3
    mx = jnp.max(jnp.where(mask, xb, NEG_INF), axis=0, keepdims=True)
    r = r1_ref[...] + r2_ref[...] + jnp.concatenate([sm, mx], axis=1)
    h = jnp.maximum(
        jnp.dot(r, m1_ref[...], preferred_element_type=jnp.float32)
        + b1_ref[...], 0.0)
    h = jnp.maximum(
        jnp.dot(h, m2_ref[...], preferred_element_type=jnp.float32)
        + b2_ref[...], 0.0)
    y = (jnp.dot(h, m3_ref[...], preferred_element_type=jnp.float32)
         + b3_ref[...])
    ymax = jnp.max(y, axis=1, keepdims=True)
    e = jnp.exp(y - ymax)
    lse = jnp.log(jnp.sum(e, axis=1, keepdims=True))
    o_ref[...] = y - ymax - lse

  return pl.pallas_call(
      body,
      out_shape=jax.ShapeDtypeStruct((1, 10), jnp.float32),
  )(xnew3, r1, r2, M1, bm1, M2, bm2, M3, bm3)


def _stage_pool(g, tu, scp, brel, src3, dst3, n, k, CH, NP, NPn, last, SB):
  u2d = tu[:, 1].reshape(NW, CH)
  score, skey, gate, prm = _tc_selprep(
      scp.reshape(NW, NW, CH), u2d, brel.reshape(1, 1), n, k, CH)
  del score
  snx, xnew = _sc_select(skey, gate, prm, g, CH, NPn, k, SB)
  if last:
    return xnew, None, None, None
  src3n, dst3n, degn = _sc_reindex(src3, dst3, snx.reshape(NP), NP, NPn, k)
  return xnew, src3n, dst3n, degn


def kernel(x, edge_index, W1, b1, Wrel1, brel1, Wroot1, W2, b2, Wrel2, brel2,
           Wroot2, W3, b3, Wrel3, brel3, Wroot3, M1, bm1, M2, bm2, M3, bm3):
  src3 = edge_index[0].astype(jnp.int32).reshape(NW, NEB, EB)
  dst3 = edge_index[1].astype(jnp.int32).reshape(NW, NEB, EB)
  xp = jnp.pad(x, ((0, NP1 - N1), (0, 0)))

  wru1 = jnp.concatenate([Wrel1, Wroot1], axis=1)
  wru2 = jnp.concatenate([Wrel2, Wroot2], axis=1)
  wru3 = jnp.concatenate([Wrel3, Wroot3], axis=1)

  deg1 = _sc_deg1(dst3)
  h1 = _tc_pre(xp, W1, deg1, N1, NP1, False)
  acc1 = _sc_msg(h1, src3, dst3, NP1)
  g1, tu1 = _tc_mid(acc1, h1, deg1, b1.reshape(1, D), wru1, N1, NP1)
  scp1 = _sc_score(tu1[:, 0], src3, dst3, NP1)
  xnew1, src3b, dst3b, deg2 = _stage_pool(
      g1, tu1, scp1, brel1, src3, dst3, N1, K1, CH1, NP1, NP2, False, 80)

  h2, r1 = _tc_pre(xnew1, W2, deg2, K1, NP2, True)
  acc2 = _sc_msg(h2, src3b, dst3b, NP2)
  g2, tu2 = _tc_mid(acc2, h2, deg2, b2.reshape(1, D), wru2, K1, NP2)
  scp2 = _sc_score(tu2[:, 0], src3b, dst3b, NP2)
  xnew2, src3c, dst3c, deg3 = _stage_pool(
      g2, tu2, scp2, brel2, src3b, dst3b, K1, K2, CH2, NP2, NP3, False, 96)

  h3, r2 = _tc_pre(xnew2, W3, deg3, K2, NP3, True)
  acc3 = _sc_msg(h3, src3c, dst3c, NP3)
  g3, tu3 = _tc_mid(acc3, h3, deg3, b3.reshape(1, D), wru3, K2, NP3)
  scp3 = _sc_score(tu3[:, 0], src3c, dst3c, NP3)
  xnew3, _, _, _ = _stage_pool(
      g3, tu3, scp3, brel3, src3c, dst3c, K2, K3, CH3, NP3, NP4, True, 112)

  return _tc_head(xnew3, r1, r2, M1, bm1.reshape(1, D), M2,
                  bm2.reshape(1, D // 2), M3, bm3.reshape(1, 10))


# speedup vs baseline: 26.4778x; 1.5156x over previous
import functools

import jax
import jax.numpy as jnp
from jax import lax
from jax.experimental import pallas as pl
from jax.experimental.pallas import tpu as pltpu
from jax.experimental.pallas import tpu_sc as plsc

D = 128
N1 = 10000
E = 320000
K1, K2, K3 = 5000, 2500, 1250

NW = 32
NS = 16
EPT = E // NW
EB = 80
NEB = EPT // EB

CH1, NP1 = 320, 32 * 320
CH2, NP2 = 192, 32 * 192
CH3, NP3 = 112, 32 * 112
CH4, NP4 = 80, 32 * 80

BR = 256
TRD = 1024
NEG_INF = float("-inf")

_mesh = functools.partial(
    plsc.VectorSubcoreMesh, core_axis_name="c", subcore_axis_name="s",
    num_cores=2, num_subcores=NS)


def _wid():
  return lax.axis_index("c") * NS + lax.axis_index("s")


def _splat(v, lane):
  idx = jnp.zeros((16,), jnp.int32) + lane
  return v.at[idx].get(mode="promise_in_bounds")


def _zero_1d(ref, nwords):
  z = jnp.zeros((16,), ref.dtype)
  def body(i, _):
    ref[pl.ds(i * 16, 16)] = z
    return 0
  lax.fori_loop(0, nwords // 16, body, 0)


def _sc_deg1(dst3):
  def body(dst_hbm, out_hbm, dbuf, degv):
    w = _wid()
    pltpu.sync_copy(dst_hbm.at[w], dbuf)
    _zero_1d(degv, NP1)
    ones = jnp.ones((16,), jnp.float32)
    def bstep(b, _):
      for j in range(EB // 16):
        d16 = dbuf[b, pl.ds(j * 16, 16)]
        plsc.addupdate_scatter(degv, [d16], ones)
      return 0
    lax.fori_loop(0, NEB, bstep, 0)
    pltpu.sync_copy(degv, out_hbm.at[w])

  return pl.kernel(
      body,
      out_type=jax.ShapeDtypeStruct((NW, NP1), jnp.float32),
      mesh=_mesh(),
      compiler_params=pltpu.CompilerParams(needs_layout_passes=False),
      scratch_types=[
          pltpu.VMEM((NEB, EB), jnp.int32),
          pltpu.VMEM((NP1,), jnp.float32),
      ],
  )(dst3)


def _sc_msg(h, src3, dst3, NP):
  ZB = 16
  RPT = NP // NS
  EBM = 25
  NBUF = 8
  CPB = 40
  NCH = EPT // (CPB * EBM)

  def body(h_hbm, src_hbm, dst_hbm, out_hbm, sbuf, dbuf, rows, zrows, accS,
           *sems):
    gsems = sems[:NBUF]
    ssems = sems[NBUF:]
    c = lax.axis_index("c")
    s = lax.axis_index("s")
    w = c * NS + s
    z = jnp.zeros((16,), jnp.float32)
    def zrow(i, _):
      r = i // 8
      l = i - r * 8
      zrows[r, pl.ds(l * 16, 16)] = z
      return 0
    lax.fori_loop(0, ZB * 8, zrow, 0)
    def zcp(j, _):
      pltpu.sync_copy(zrows, accS.at[pl.ds(s * RPT + j * ZB, ZB)])
      return 0
    lax.fori_loop(0, RPT // ZB, zcp, 0)
    plsc.subcore_barrier()
    def chunk(ci, _):
      pltpu.sync_copy(src_hbm.at[w, ci], sbuf)
      pltpu.sync_copy(dst_hbm.at[w, ci], dbuf)
      def rnd(r, _):
        for i in range(NBUF):
          j = r * NBUF + i
          @pl.when(r > 0)
          def _():
            pltpu.make_async_copy(rows.at[i], accS.at[dbuf.at[j]],
                                  ssems[i]).wait()
          pltpu.async_copy(h_hbm.at[sbuf.at[j]], rows.at[i], gsems[i])
        for i in range(NBUF):
          j = r * NBUF + i
          pltpu.make_async_copy(h_hbm.at[sbuf.at[j]], rows.at[i],
                                gsems[i]).wait()
          pltpu.async_copy(rows.at[i], accS.at[dbuf.at[j]], ssems[i],
                           add=True)
        return 0
      lax.fori_loop(0, CPB // NBUF, rnd, 0)
      for i in range(NBUF):
        pltpu.make_async_copy(rows.at[i], accS.at[dbuf.at[CPB - NBUF + i]],
                              ssems[i]).wait()
      return 0
    lax.fori_loop(0, NCH, chunk, 0)
    plsc.subcore_barrier()
    def ocp(j, _):
      pltpu.sync_copy(accS.at[pl.ds(s * RPT + j * ZB, ZB)],
                      out_hbm.at[c, pl.ds(s * RPT + j * ZB, ZB)])
      return 0
    lax.fori_loop(0, RPT // ZB, ocp, 0)

  return pl.kernel(
      body,
      out_type=jax.ShapeDtypeStruct((2, NP, D), jnp.float32),
      mesh=_mesh(),
      compiler_params=pltpu.CompilerParams(needs_layout_passes=False),
      scratch_types=[
          pltpu.VMEM((CPB, EBM), jnp.int32),
          pltpu.VMEM((CPB, EBM), jnp.int32),
          pltpu.VMEM((NBUF, EBM, D), jnp.float32),
          pltpu.VMEM((ZB, D), jnp.float32),
          pltpu.VMEM_SHARED((NP, D), jnp.float32),
      ] + [pltpu.SemaphoreType.DMA] * 16,
  )(h, src3.reshape(NW, NCH, CPB, EBM), dst3.reshape(NW, NCH, CPB, EBM))


def _sc_score(t, src3, dst3, NP):
  def body(t_hbm, src_hbm, dst_hbm, out_hbm, tv, sbuf, dbuf, scv):
    w = _wid()
    pltpu.sync_copy(t_hbm, tv)
    pltpu.sync_copy(src_hbm.at[w], sbuf)
    pltpu.sync_copy(dst_hbm.at[w], dbuf)
    _zero_1d(scv, NP)
    def bstep(b, _):
      for j in range(EB // 16):
        s16 = sbuf[b, pl.ds(j * 16, 16)]
        d16 = dbuf[b, pl.ds(j * 16, 16)]
        tv16 = plsc.load_gather(tv, [s16])
        plsc.addupdate_scatter(scv, [d16], tv16)
      return 0
    lax.fori_loop(0, NEB, bstep, 0)
    pltpu.sync_copy(scv, out_hbm.at[w])

  return pl.kernel(
      body,
      out_type=jax.ShapeDtypeStruct((NW, NP), jnp.float32),
      mesh=_mesh(),
      compiler_params=pltpu.CompilerParams(needs_layout_passes=False),
      scratch_types=[
          pltpu.VMEM((NP,), jnp.float32),
          pltpu.VMEM((NEB, EB), jnp.int32),
          pltpu.VMEM((NEB, EB), jnp.int32),
          pltpu.VMEM((NP,), jnp.float32),
      ],
  )(t, src3, dst3)


def _sc_select(skey, gate, params, g, CH, NPn, trash, SB):
  NBX = CH // SB
  SG = SB // 16

  def body(skey_hbm, gate_hbm, prm_hbm, g_hbm, snx_hbm, xnew_hbm,
           skv, gatev, prmv, rows, selv, nidx2d, sem):
    w = _wid()
    pltpu.sync_copy(skey_hbm.at[w], skv)
    pltpu.sync_copy(gate_hbm.at[w], gatev)
    pltpu.sync_copy(prm_hbm.at[w], prmv)
    pltpu.sync_copy(g_hbm.at[pl.ds(w * CH, CH)], rows)
    prm = prmv[...]
    eq_carry = _splat(prm, 0)
    pos_carry = _splat(prm, 1)
    rsp = _splat(prm, 2)
    tkey = _splat(prm, 3)
    trash_v = jnp.zeros((16,), jnp.int32) + trash
    neg1 = jnp.zeros((16,), jnp.int32) - 1
    last = jnp.zeros((16,), jnp.int32) + 15
    iota16 = lax.iota(jnp.int32, 16)
    for v in range(CH // 16):
      spread = trash_v + ((w * CH + v * 16 + iota16) & (TRD - 1))
      sk = skv[pl.ds(v * 16, 16)]
      gt = sk > tkey
      eqm = sk == tkey
      eqi = eqm.astype(jnp.int32)
      cse = plsc.cumsum(eqi)
      rank = eq_carry + (cse - eqi)
      sel16 = gt | (eqm & (rank < rsp))
      seli = sel16.astype(jnp.int32)
      css = plsc.cumsum(seli)
      npos = pos_carry + (css - seli)
      nidx2d[v // SG, pl.ds((v % SG) * 16, 16)] = jnp.where(
          sel16, npos, spread)
      selv[pl.ds(v * 16, 16)] = jnp.where(sel16, npos, neg1)
      eq_carry = eq_carry + cse.at[last].get(mode="promise_in_bounds")
      pos_carry = pos_carry + css.at[last].get(mode="promise_in_bounds")
    def rscale(r, _):
      v = r // 16
      lane = r - v * 16
      gsp = _splat(gatev[pl.ds(v * 16, 16)], lane)
      for f in range(D // 16):
        rows[r, pl.ds(f * 16, 16)] = rows[r, pl.ds(f * 16, 16)] * gsp
      return 0
    lax.fori_loop(0, CH, rscale, 0)
    for bx in range(NBX):
      pltpu.async_copy(rows.at[pl.ds(bx * SB, SB)],
                       xnew_hbm.at[nidx2d.at[bx]], sem).wait()
    pltpu.sync_copy(selv, snx_hbm.at[w])

  return pl.kernel(
      body,
      out_type=(
          jax.ShapeDtypeStruct((NW, CH), jnp.int32),
          jax.ShapeDtypeStruct((NPn, D), jnp.float32),
      ),
      mesh=_mesh(),
      compiler_params=pltpu.CompilerParams(needs_layout_passes=False),
      scratch_types=[
          pltpu.VMEM((CH,), jnp.int32),
          pltpu.VMEM((CH,), jnp.float32),
          pltpu.VMEM((16,), jnp.int32),
          pltpu.VMEM((CH, D), jnp.float32),
          pltpu.VMEM((CH,), jnp.int32),
          pltpu.VMEM((NBX, SB), jnp.int32),
          pltpu.SemaphoreType.DMA,
      ],
  )(skey, gate, params, g)


def _sc_reindex(src3, dst3, snx, NP, NPn, trash):
  def body(src_hbm, dst_hbm, snx_hbm, srcn_hbm, dstn_hbm, deg_hbm,
           snxv, sbuf, dbuf, sobuf, dobuf, degv):
    w = _wid()
    pltpu.sync_copy(snx_hbm, snxv)
    pltpu.sync_copy(src_hbm.at[w], sbuf)
    pltpu.sync_copy(dst_hbm.at[w], dbuf)
    _zero_1d(degv, NPn)
    ones = jnp.ones((16,), jnp.float32)
    trash_v = jnp.zeros((16,), jnp.int32) + trash
    iota16 = lax.iota(jnp.int32, 16)
    def bstep(b, _):
      for j in range(EB // 16):
        s16 = sbuf[b, pl.ds(j * 16, 16)]
        d16 = dbuf[b, pl.ds(j * 16, 16)]
        gs = plsc.load_gather(snxv, [s16])
        gd = plsc.load_gather(snxv, [d16])
        valid = (gs >= 0) & (gd >= 0)
        spread = trash_v + ((b * EB + j * 16 + iota16) & (TRD - 1))
        nd = jnp.where(valid, gd, spread)
        sobuf[b, pl.ds(j * 16, 16)] = jnp.where(valid, gs, spread)
        dobuf[b, pl.ds(j * 16, 16)] = nd
        plsc.addupdate_scatter(degv, [nd], ones)
      return 0
    lax.fori_loop(0, NEB, bstep, 0)
    pltpu.sync_copy(sobuf, srcn_hbm.at[w])
    pltpu.sync_copy(dobuf, dstn_hbm.at[w])
    pltpu.sync_copy(degv, deg_hbm.at[w])

  return pl.kernel(
      body,
      out_type=(
          jax.ShapeDtypeStruct((NW, NEB, EB), jnp.int32),
          jax.ShapeDtypeStruct((NW, NEB, EB), jnp.int32),
          jax.ShapeDtypeStruct((NW, NPn), jnp.float32),
      ),
      mesh=_mesh(),
      compiler_params=pltpu.CompilerParams(needs_layout_passes=False),
      scratch_types=[
          pltpu.VMEM((NP,), jnp.int32),
          pltpu.VMEM((NEB, EB), jnp.int32),
          pltpu.VMEM((NEB, EB), jnp.int32),
          pltpu.VMEM((NEB, EB), jnp.int32),
          pltpu.VMEM((NEB, EB), jnp.int32),
          pltpu.VMEM((NPn,), jnp.float32),
      ],
  )(src3, dst3, snx)


def _tc_pre(x, W, deg_part, n, NP, with_readout):
  RB = NP // BR

  def body(*refs):
    if with_readout:
      x_ref, w_ref, dp_ref, h_ref, r_ref = refs
    else:
      x_ref, w_ref, dp_ref, h_ref = refs
    i = pl.program_id(0)
    deg = jnp.sum(dp_ref[...], axis=0) + 1.0
    dinv = lax.rsqrt(deg)
    rowid = i * BR + lax.broadcasted_iota(jnp.int32, (BR, D), 0)
    mask = rowid < n
    xb = x_ref[...]
    h = jnp.dot(xb, w_ref[...], preferred_element_type=jnp.float32)
    h_ref[...] = jnp.where(mask, h * dinv[:, None], 0.0)
    if with_readout:
      @pl.when(i == 0)
      def _():
        r_ref[...] = jnp.concatenate(
            [jnp.zeros((1, D), jnp.float32),
             jnp.full((1, D), NEG_INF, jnp.float32)], axis=1)
      sm = jnp.sum(jnp.where(mask, xb, 0.0), axis=0, keepdims=True) / n
      mx = jnp.max(jnp.where(mask, xb, NEG_INF), axis=0, keepdims=True)
      r_ref[:, 0:D] = r_ref[:, 0:D] + sm
      r_ref[:, D:2 * D] = jnp.maximum(r_ref[:, D:2 * D], mx)

  out_shape = [jax.ShapeDtypeStruct((NP, D), jnp.float32)]
  out_specs = [pl.BlockSpec((BR, D), lambda i: (i, 0))]
  if with_readout:
    out_shape.append(jax.ShapeDtypeStruct((1, 2 * D), jnp.float32))
    out_specs.append(pl.BlockSpec((1, 2 * D), lambda i: (0, 0)))
  res = pl.pallas_call(
      body,
      grid=(RB,),
      in_specs=[
          pl.BlockSpec((BR, D), lambda i: (i, 0)),
          pl.BlockSpec((D, D), lambda i: (0, 0)),
          pl.BlockSpec((NW, BR), lambda i: (0, i)),
      ],
      out_specs=out_specs,
      out_shape=out_shape,
  )(x, W, deg_part)
  return res if with_readout else res[0]


def _tc_mid(acc2, h, deg_part, b, wru, n, NP):
  RB = NP // BR

  def body(a_ref, h_ref, dp_ref, b_ref, w_ref, g_ref, tu_ref):
    i = pl.program_id(0)
    deg = jnp.sum(dp_ref[...], axis=0) + 1.0
    dinv = lax.rsqrt(deg)
    rowid = i * BR + lax.broadcasted_iota(jnp.int32, (BR, D), 0)
    mask = rowid < n
    acc = a_ref[0] + a_ref[1] + h_ref[...]
    g = jnp.maximum(acc * dinv[:, None] + b_ref[...], 0.0)
    g = jnp.where(mask, g, 0.0)
    g_ref[...] = g
    tu_ref[...] = jnp.dot(g, w_ref[...], preferred_element_type=jnp.float32)

  return pl.pallas_call(
      body,
      grid=(RB,),
      in_specs=[
          pl.BlockSpec((2, BR, D), lambda i: (0, i, 0)),
          pl.BlockSpec((BR, D), lambda i: (i, 0)),
          pl.BlockSpec((NW, BR), lambda i: (0, i)),
          pl.BlockSpec((1, D), lambda i: (0, 0)),
          pl.BlockSpec((D, 2), lambda i: (0, 0)),
      ],
      out_specs=[
          pl.BlockSpec((BR, D), lambda i: (i, 0)),
          pl.BlockSpec((BR, 2), lambda i: (i, 0)),
      ],
      out_shape=[
          jax.ShapeDtypeStruct((NP, D), jnp.float32),
          jax.ShapeDtypeStruct((NP, 2), jnp.float32),
      ],
  )(acc2, h, deg_part, b, wru)


def _tc_selprep(scp, u2d, brel, n, k, CH):
  def body(scp_ref, u_ref, br_ref, score_ref, skey_ref, gate_ref, prm_ref):
    INT_MIN = jnp.int32(-2147483648)
    a = pl.program_id(0)

    @pl.when(a == 0)
    def _():
      score_ref[...] = jnp.zeros((NW, CH), jnp.float32)
    score_ref[...] = score_ref[...] + scp_ref[0]

    @pl.when(a == NW - 1)
    def _():
      s = score_ref[...] + u_ref[...] + br_ref[0, 0]
      flat = (lax.broadcasted_iota(jnp.int32, (NW, CH), 0) * CH
              + lax.broadcasted_iota(jnp.int32, (NW, CH), 1))
      s = jnp.where(flat < n, s, NEG_INF)
      score_ref[...] = s
      bits = lax.bitcast_convert_type(s, jnp.int32)
      m = lax.shift_right_arithmetic(bits, 31)
      skey = bits ^ (m & jnp.int32(0x7FFFFFFF))
      skey_ref[...] = skey
      kf = jnp.float32(k)

      def bstep(i, tu):
        bit = 31 - i
        cand = tu | lax.shift_left(jnp.int32(1), bit)
        thr = cand ^ INT_MIN
        c = jnp.sum((skey >= thr).astype(jnp.float32))
        return jnp.where(c >= kf, cand, tu)

      tu = lax.fori_loop(0, 32, bstep, jnp.int32(0))
      tkey = tu ^ INT_MIN
      cgt = jnp.sum((skey > tkey).astype(jnp.float32))
      r = kf - cgt
      gt_t = jnp.sum((skey > tkey).astype(jnp.float32), axis=1, keepdims=True)
      eq_t = jnp.sum((skey == tkey).astype(jnp.float32), axis=1, keepdims=True)
      tri = (lax.broadcasted_iota(jnp.int32, (NW, NW), 0)
             > lax.broadcasted_iota(jnp.int32, (NW, NW), 1)).astype(jnp.float32)
      gt_b = jnp.dot(tri, gt_t, preferred_element_type=jnp.float32)
      eq_b = jnp.dot(tri, eq_t, preferred_element_type=jnp.float32)
      base = gt_b + jnp.minimum(eq_b, r)
      prm = jnp.concatenate([
          eq_b.astype(jnp.int32),
          base.astype(jnp.int32),
          jnp.full((NW, 1), r, jnp.float32).astype(jnp.int32),
          jnp.full((NW, 1), tkey, jnp.int32),
          jnp.zeros((NW, 12), jnp.int32),
      ], axis=1)
      prm_ref[...] = prm
      gate_ref[...] = jnp.tanh(s)

  return pl.pallas_call(
      body,
      grid=(NW,),
      in_specs=[
          pl.BlockSpec((1, NW, CH), lambda a: (a, 0, 0)),
          pl.BlockSpec((NW, CH), lambda a: (0, 0)),
          pl.BlockSpec((1, 1), lambda a: (0, 0)),
      ],
      out_specs=[
          pl.BlockSpec((NW, CH), lambda a: (0, 0)),
          pl.BlockSpec((NW, CH), lambda a: (0, 0)),
          pl.BlockSpec((NW, CH), lambda a: (0, 0)),
          pl.BlockSpec((NW, 16), lambda a: (0, 0)),
      ],
      out_shape=[
          jax.ShapeDtypeStruct((NW, CH), jnp.float32),
          jax.ShapeDtypeStruct((NW, CH), jnp.int32),
          jax.ShapeDtypeStruct((NW, CH), jnp.float32),
          jax.ShapeDtypeStruct((NW, 16), jnp.int32),
      ],
  )(scp, u2d, brel)


def _tc_head(xnew3, r1, r2, M1, bm1, M2, bm2, M3, bm3):
  def body(x_ref, r1_ref, r2_ref, m1_ref, b1_ref, m2_ref, b2_ref, m3_ref,
           b3_ref, o_ref):
    mask = lax.broadcasted_iota(jnp.int32, (NP4, D), 0) < K3
    xb = x_ref[...]
    sm = jnp.sum(jnp.where(mask, xb, 0.0), axis=0, keepdims=True) / K3
    mx = jnp.max(jnp.where(mask, xb, NEG_INF), axis=0, keepdims=True)
    r = r1_ref[...] + r2_ref[...] + jnp.concatenate([sm, mx], axis=1)
    h = jnp.maximum(
        jnp.dot(r, m1_ref[...], preferred_element_type=jnp.float32)
        + b1_ref[...], 0.0)
    h = jnp.maximum(
        jnp.dot(h, m2_ref[...], preferred_element_type=jnp.float32)
        + b2_ref[...], 0.0)
    y = (jnp.dot(h, m3_ref[...], preferred_element_type=jnp.float32)
         + b3_ref[...])
    ymax = jnp.max(y, axis=1, keepdims=True)
    e = jnp.exp(y - ymax)
    lse = jnp.log(jnp.sum(e, axis=1, keepdims=True))
    o_ref[...] = y - ymax - lse

  return pl.pallas_call(
      body,
      out_shape=jax.ShapeDtypeStruct((1, 10), jnp.float32),
  )(xnew3, r1, r2, M1, bm1, M2, bm2, M3, bm3)


def _stage_pool(g, tu, scp, brel, src3, dst3, n, k, CH, NP, NPn, last, SB):
  u2d = tu[:, 1].reshape(NW, CH)
  score, skey, gate, prm = _tc_selprep(
      scp.reshape(NW, NW, CH), u2d, brel.reshape(1, 1), n, k, CH)
  del score
  snx, xnew = _sc_select(skey, gate, prm, g, CH, NPn, k, SB)
  if last:
    return xnew, None, None, None
  src3n, dst3n, degn = _sc_reindex(src3, dst3, snx.reshape(NP), NP, NPn, k)
  return xnew, src3n, dst3n, degn


def kernel(x, edge_index, W1, b1, Wrel1, brel1, Wroot1, W2, b2, Wrel2, brel2,
           Wroot2, W3, b3, Wrel3, brel3, Wroot3, M1, bm1, M2, bm2, M3, bm3):
  src3 = edge_index[0].astype(jnp.int32).reshape(NW, NEB, EB)
  dst3 = edge_index[1].astype(jnp.int32).reshape(NW, NEB, EB)
  xp = jnp.pad(x, ((0, NP1 - N1), (0, 0)))

  wru1 = jnp.concatenate([Wrel1, Wroot1], axis=1)
  wru2 = jnp.concatenate([Wrel2, Wroot2], axis=1)
  wru3 = jnp.concatenate([Wrel3, Wroot3], axis=1)

  deg1 = _sc_deg1(dst3)
  h1 = _tc_pre(xp, W1, deg1, N1, NP1, False)
  acc1 = _sc_msg(h1, src3, dst3, NP1)
  g1, tu1 = _tc_mid(acc1, h1, deg1, b1.reshape(1, D), wru1, N1, NP1)
  scp1 = _sc_score(tu1[:, 0], src3, dst3, NP1)
  xnew1, src3b, dst3b, deg2 = _stage_pool(
      g1, tu1, scp1, brel1, src3, dst3, N1, K1, CH1, NP1, NP2, False, 80)

  h2, r1 = _tc_pre(xnew1, W2, deg2, K1, NP2, True)
  acc2 = _sc_msg(h2, src3b, dst3b, NP2)
  g2, tu2 = _tc_mid(acc2, h2, deg2, b2.reshape(1, D), wru2, K1, NP2)
  scp2 = _sc_score(tu2[:, 0], src3b, dst3b, NP2)
  xnew2, src3c, dst3c, deg3 = _stage_pool(
      g2, tu2, scp2, brel2, src3b, dst3b, K1, K2, CH2, NP2, NP3, False, 96)

  h3, r2 = _tc_pre(xnew2, W3, deg3, K2, NP3, True)
  acc3 = _sc_msg(h3, src3c, dst3c, NP3)
  g3, tu3 = _tc_mid(acc3, h3, deg3, b3.reshape(1, D), wru3, K2, NP3)
  scp3 = _sc_score(tu3[:, 0], src3c, dst3c, NP3)
  xnew3, _, _, _ = _stage_pool(
      g3, tu3, scp3, brel3, src3c, dst3c, K2, K3, CH3, NP3, NP4, True, 112)

  return _tc_head(xnew3, r1, r2, M1, bm1.reshape(1, D), M2,
                  bm2.reshape(1, D // 2), M3, bm3.reshape(1, 10))
